# Initial kernel scaffold; baseline (speedup 1.0000x reference)
import functools

import jax
import jax.numpy as jnp
from jax import lax
from jax.experimental import pallas as pl
from jax.experimental.pallas import tpu as pltpu
from jax.experimental.pallas import tpu_sc as plsc

BATCH = 4096
HIST = 200
D = 32
OUT_L = HIST + 1
NC = 2
NS = 16
NW = NC * NS
B_PER_W = BATCH // NW


def _sc_embed(val, table, global_token):
    mesh = plsc.VectorSubcoreMesh(core_axis_name="c", subcore_axis_name="s")

    @functools.partial(
        pl.kernel,
        mesh=mesh,
        out_type=jax.ShapeDtypeStruct((BATCH, OUT_L, D), jnp.float32),
        scratch_types=[
            pltpu.VMEM((HIST,), jnp.int32),
            pltpu.VMEM((OUT_L, D), jnp.float32),
            pltpu.SemaphoreType.DMA,
        ],
    )
    def k(val_hbm, table_hbm, gt_hbm, out_hbm, idx_v, rows_v, sem):
        wid = lax.axis_index("s") * NC + lax.axis_index("c")
        b0 = wid * B_PER_W
        pltpu.sync_copy(gt_hbm, rows_v.at[pl.ds(0, 1)])

        def body(i, carry):
            b = b0 + i
            pltpu.sync_copy(val_hbm.at[b], idx_v)
            pltpu.async_copy(table_hbm.at[idx_v], rows_v.at[pl.ds(1, HIST)], sem).wait()
            pltpu.sync_copy(rows_v, out_hbm.at[b])
            return carry

        lax.fori_loop(0, B_PER_W, body, 0)

    return k(val, table, global_token)


def kernel(val, table, global_token):
    return _sc_embed(val.astype(jnp.int32), table, global_token)


# speedup vs baseline: 1.4449x; 1.4449x over previous
import functools

import jax
import jax.numpy as jnp
from jax import lax
from jax.experimental import pallas as pl
from jax.experimental.pallas import tpu as pltpu
from jax.experimental.pallas import tpu_sc as plsc

BATCH = 4096
HIST = 200
D = 32
OUT_L = HIST + 1
NC = 2
NS = 16
NW = NC * NS
B_PER_W = BATCH // NW


def _sc_embed(val, table, global_token):
    mesh = plsc.VectorSubcoreMesh(core_axis_name="c", subcore_axis_name="s")

    @functools.partial(
        pl.kernel,
        mesh=mesh,
        out_type=jax.ShapeDtypeStruct((BATCH, OUT_L, D), jnp.float32),
        scratch_types=[
            pltpu.VMEM((HIST,), jnp.int32),
            pltpu.VMEM((OUT_L, D), jnp.float32),
            pltpu.SemaphoreType.DMA,
        ],
        compiler_params=pltpu.CompilerParams(use_tc_tiling_on_sc=False),
    )
    def k(val_hbm, table_hbm, gt_hbm, out_hbm, idx_v, rows_v, sem):
        wid = lax.axis_index("s") * NC + lax.axis_index("c")
        b0 = wid * B_PER_W
        pltpu.sync_copy(gt_hbm, rows_v.at[pl.ds(0, 1)])

        def body(i, carry):
            b = b0 + i
            pltpu.sync_copy(val_hbm.at[b], idx_v)
            pltpu.async_copy(table_hbm.at[idx_v], rows_v.at[pl.ds(1, HIST)], sem).wait()
            pltpu.sync_copy(rows_v, out_hbm.at[b])
            return carry

        lax.fori_loop(0, B_PER_W, body, 0)

    return k(val, table, global_token)


def kernel(val, table, global_token):
    return _sc_embed(val.astype(jnp.int32), table, global_token)


# speedup vs baseline: 1.6818x; 1.1640x over previous
import functools

import jax
import jax.numpy as jnp
from jax import lax
from jax.experimental import pallas as pl
from jax.experimental.pallas import tpu as pltpu
from jax.experimental.pallas import tpu_sc as plsc

BATCH = 4096
HIST = 200
D = 32
OUT_L = HIST + 1
NC = 2
NS = 16
NW = NC * NS
B_PER_W = BATCH // NW
NB = 4
CHUNKS = B_PER_W // NB


def _sc_embed(val, table, global_token):
    mesh = plsc.VectorSubcoreMesh(core_axis_name="c", subcore_axis_name="s")

    @functools.partial(
        pl.kernel,
        mesh=mesh,
        out_type=jax.ShapeDtypeStruct((BATCH, OUT_L, D), jnp.float32),
        scratch_types=[
            pltpu.VMEM((B_PER_W, HIST), jnp.int32),
            pltpu.VMEM((NB, OUT_L, D), jnp.float32),
            pltpu.VMEM((NB, OUT_L, D), jnp.float32),
            pltpu.SemaphoreType.DMA,
            pltpu.SemaphoreType.DMA,
            pltpu.SemaphoreType.DMA,
            pltpu.SemaphoreType.DMA,
        ],
        compiler_params=pltpu.CompilerParams(use_tc_tiling_on_sc=False),
    )
    def k(val_hbm, table_hbm, gt_hbm, out_hbm,
          idx_all, stg0, stg1, gsem0, gsem1, wsem0, wsem1):
        wid = lax.axis_index("s") * NC + lax.axis_index("c")
        b0 = wid * B_PER_W

        pltpu.sync_copy(val_hbm.at[pl.ds(b0, B_PER_W)], idx_all)
        for stg in (stg0, stg1):
            for j in range(NB):
                pltpu.sync_copy(gt_hbm, stg.at[j, pl.ds(0, 1)])

        def g_copy(c, stg, gsem, j):
            return pltpu.make_async_copy(
                table_hbm.at[idx_all.at[c * NB + j]],
                stg.at[j, pl.ds(1, HIST)],
                gsem,
            )

        def w_copy(c, stg, wsem):
            return pltpu.make_async_copy(
                stg, out_hbm.at[pl.ds(b0 + c * NB, NB)], wsem)

        def fire_g(c, stg, gsem):
            for j in range(NB):
                g_copy(c, stg, gsem, j).start()

        def drain_g(c, stg, gsem):
            for j in range(NB):
                g_copy(c, stg, gsem, j).wait()

        def step(c, stg, gsem, wsem, prefetch):
            drain_g(c, stg, gsem)
            w_copy(c, stg, wsem).start()
            if prefetch:
                w_copy(c, stg, wsem).wait()
                fire_g(c + 2, stg, gsem)

        fire_g(0, stg0, gsem0)
        fire_g(1, stg1, gsem1)

        def body(p, carry):
            step(2 * p, stg0, gsem0, wsem0, True)
            step(2 * p + 1, stg1, gsem1, wsem1, True)
            return carry

        lax.fori_loop(0, CHUNKS // 2 - 1, body, 0)
        step(CHUNKS - 2, stg0, gsem0, wsem0, False)
        step(CHUNKS - 1, stg1, gsem1, wsem1, False)
        w_copy(CHUNKS - 2, stg0, wsem0).wait()
        w_copy(CHUNKS - 1, stg1, wsem1).wait()

    return k(val, table, global_token)


def kernel(val, table, global_token):
    return _sc_embed(val.astype(jnp.int32), table, global_token)
